# baseline (speedup 1.0000x reference)
import functools

import jax
import jax.numpy as jnp
from jax import lax
from jax.experimental import pallas as pl
from jax.experimental.pallas import tpu as pltpu
from jax.experimental.pallas import tpu_sc as plsc

BATCH = 16384
EMBED = 32
HIDDEN = 64
BLK = 2048
LANES = 128
NBUF = 8


def _sc_gather(user_id, item_id, ut_t, it_t):
    info = plsc.get_sparse_core_info()
    nc, ns = info.num_cores, info.num_subcores
    nw = nc * ns
    b_per_w = BATCH // nw
    g16 = b_per_w // 16
    mesh = plsc.VectorSubcoreMesh(core_axis_name="c", subcore_axis_name="s")

    @functools.partial(
        pl.kernel,
        mesh=mesh,
        compiler_params=pltpu.CompilerParams(
            disable_bounds_checks=True, needs_layout_passes=False),
        out_type=jax.ShapeDtypeStruct((2 * EMBED * BATCH,), jnp.float32),
        scratch_types=[
            pltpu.VMEM((b_per_w,), jnp.int32),
            pltpu.VMEM((b_per_w,), jnp.int32),
            pltpu.VMEM((2 * EMBED * b_per_w,), jnp.float32),
        ]
        + [pltpu.VMEM((EMBED, LANES), jnp.float32) for _ in range(2 * NBUF)]
        + [pltpu.SemaphoreType.DMA for _ in range(2 * NBUF)],
    )
    def gather_k(uid_hbm, iid_hbm, ut_hbm, it_hbm, xt_hbm,
                 uidx_v, iidx_v, xt_v, *stage_and_sems):
        stage = stage_and_sems[:2 * NBUF]
        sems = stage_and_sems[2 * NBUF:]
        wid = lax.axis_index("s") * nc + lax.axis_index("c")
        base = wid * b_per_w
        pltpu.sync_copy(uid_hbm.at[pl.ds(base, b_per_w)], uidx_v)
        pltpu.sync_copy(iid_hbm.at[pl.ds(base, b_per_w)], iidx_v)

        iota16 = lax.iota(jnp.int32, 16)
        xpos_lo = iota16 * b_per_w
        xpos_hi = (iota16 + 16) * b_per_w

        def fire(tbl, col, buf, sem):
            tile0 = pl.multiple_of((col >> 7) << 7, LANES)
            pltpu.async_copy(tbl.at[:, pl.ds(tile0, LANES)], buf, sem)

        def extract(e, lvec, buf, sem, xoff):
            pltpu.make_async_copy(
                ut_hbm.at[:, pl.ds(0, LANES)], buf, sem).wait()
            lo = plsc.load_gather(buf, [iota16, lvec])
            hi = plsc.load_gather(buf, [iota16 + 16, lvec])
            ecast = jnp.full((16,), e, jnp.int32)
            plsc.store_scatter(xt_v, [xoff + xpos_lo + ecast], lo)
            plsc.store_scatter(xt_v, [xoff + xpos_hi + ecast], hi)

        @pl.loop(0, g16)
        def _(g):
            uvec = uidx_v[pl.ds(g * 16, 16)]
            ivec = iidx_v[pl.ds(g * 16, 16)]
            ulane = uvec & (LANES - 1)
            ilane = ivec & (LANES - 1)
            for l in range(16):
                bu = (2 * l) % (2 * NBUF)
                bi = (2 * l + 1) % (2 * NBUF)
                if l >= NBUF:
                    lp = l - NBUF
                    extract(g * 16 + lp,
                            jnp.full((16,), ulane[lp], jnp.int32),
                            stage[bu], sems[bu], 0)
                    extract(g * 16 + lp,
                            jnp.full((16,), ilane[lp], jnp.int32),
                            stage[bi], sems[bi], EMBED * b_per_w)
                fire(ut_hbm, uvec[l], stage[bu], sems[bu])
                fire(it_hbm, ivec[l], stage[bi], sems[bi])
            for l in range(16 - NBUF, 16):
                bu = (2 * l) % (2 * NBUF)
                bi = (2 * l + 1) % (2 * NBUF)
                extract(g * 16 + l, jnp.full((16,), ulane[l], jnp.int32),
                        stage[bu], sems[bu], 0)
                extract(g * 16 + l, jnp.full((16,), ilane[l], jnp.int32),
                        stage[bi], sems[bi], EMBED * b_per_w)

        for c in range(2 * EMBED):
            pltpu.sync_copy(
                xt_v.at[pl.ds(c * b_per_w, b_per_w)],
                xt_hbm.at[pl.ds(c * BATCH + base, b_per_w)])

    return gather_k(user_id, item_id, ut_t, it_t)


def _mlp_body(xt_ref, w1_ref, b1_ref, w2_ref, b2_ref, out_ref):
    h = jnp.dot(w1_ref[...], xt_ref[...], preferred_element_type=jnp.float32)
    h = jnp.maximum(h + b1_ref[...], 0.0)
    out_ref[...] = (
        jnp.dot(w2_ref[...], h, preferred_element_type=jnp.float32)
        + b2_ref[0, 0]
    )


def _tc_mlp(xt, W1, b1_col, W2, b2_2d):
    grid = (BATCH // BLK,)
    return pl.pallas_call(
        _mlp_body,
        grid=grid,
        in_specs=[
            pl.BlockSpec((2 * EMBED, BLK), lambda i: (0, i)),
            pl.BlockSpec((HIDDEN, 2 * EMBED), lambda i: (0, 0)),
            pl.BlockSpec((HIDDEN, 1), lambda i: (0, 0)),
            pl.BlockSpec((1, HIDDEN), lambda i: (0, 0)),
            pl.BlockSpec((1, 1), lambda i: (0, 0)),
        ],
        out_specs=pl.BlockSpec((1, BLK), lambda i: (0, i)),
        out_shape=jax.ShapeDtypeStruct((1, BATCH), jnp.float32),
    )(xt, W1, b1_col, W2, b2_2d)


def kernel(user_id, item_id, user_table, item_table, W1, b1, W2, b2):
    uid = user_id.astype(jnp.int32)
    iid = item_id.astype(jnp.int32)
    xt_flat = _sc_gather(uid, iid, user_table.T, item_table.T)
    xt = xt_flat.reshape(2 * EMBED, BATCH)
    out = _tc_mlp(xt, W1, b1.reshape(HIDDEN, 1), W2, b2.reshape(1, 1))
    return out.reshape(BATCH)


# speedup vs baseline: 1.0740x; 1.0740x over previous
import functools

import jax
import jax.numpy as jnp
from jax import lax
from jax.experimental import pallas as pl
from jax.experimental.pallas import tpu as pltpu
from jax.experimental.pallas import tpu_sc as plsc

BATCH = 16384
EMBED = 32
HIDDEN = 64
BLK = 2048
LANES = 128
NBUF = 8


def _sc_gather(user_id, item_id, ut_t, it_t):
    info = plsc.get_sparse_core_info()
    nc, ns = info.num_cores, info.num_subcores
    nw = nc * ns
    b_per_w = BATCH // nw
    g16 = b_per_w // 16
    mesh = plsc.VectorSubcoreMesh(core_axis_name="c", subcore_axis_name="s")

    @functools.partial(
        pl.kernel,
        mesh=mesh,
        compiler_params=pltpu.CompilerParams(
            disable_bounds_checks=True, needs_layout_passes=False),
        out_type=jax.ShapeDtypeStruct((2 * EMBED * BATCH,), jnp.float32),
        scratch_types=[
            pltpu.VMEM((b_per_w,), jnp.int32),
            pltpu.VMEM((b_per_w,), jnp.int32),
            pltpu.VMEM((2 * EMBED * b_per_w,), jnp.float32),
        ]
        + [pltpu.VMEM((EMBED, LANES), jnp.float32) for _ in range(2 * NBUF)]
        + [pltpu.SemaphoreType.DMA for _ in range(2 * NBUF)],
    )
    def gather_k(uid_hbm, iid_hbm, ut_hbm, it_hbm, xt_hbm,
                 uidx_v, iidx_v, xt_v, *stage_and_sems):
        stage = stage_and_sems[:2 * NBUF]
        sems = stage_and_sems[2 * NBUF:]
        wid = lax.axis_index("s") * nc + lax.axis_index("c")
        base = wid * b_per_w
        pltpu.sync_copy(uid_hbm.at[pl.ds(base, b_per_w)], uidx_v)
        pltpu.sync_copy(iid_hbm.at[pl.ds(base, b_per_w)], iidx_v)

        iota16 = lax.iota(jnp.int32, 16)
        xpos_lo = iota16 * b_per_w
        xpos_hi = (iota16 + 16) * b_per_w

        def fire(tbl, col, buf, sem):
            tile0 = pl.multiple_of((col >> 7) << 7, LANES)
            pltpu.async_copy(tbl.at[:, pl.ds(tile0, LANES)], buf, sem)

        def extract(e, lvec, buf, sem, xoff):
            pltpu.make_async_copy(
                ut_hbm.at[:, pl.ds(0, LANES)], buf, sem).wait()
            lo = plsc.load_gather(buf, [iota16, lvec])
            hi = plsc.load_gather(buf, [iota16 + 16, lvec])
            ecast = jnp.full((16,), e, jnp.int32)
            plsc.store_scatter(xt_v, [xoff + xpos_lo + ecast], lo)
            plsc.store_scatter(xt_v, [xoff + xpos_hi + ecast], hi)

        stage_u, stage_i = stage[:NBUF], stage[NBUF:]
        sems_u, sems_i = sems[:NBUF], sems[NBUF:]
        xoff_i = EMBED * b_per_w

        uv0 = uidx_v[pl.ds(0, 16)]
        iv0 = iidx_v[pl.ds(0, 16)]
        for e in range(NBUF):
            fire(ut_hbm, uv0[e], stage_u[e], sems_u[e])
            fire(it_hbm, iv0[e], stage_i[e], sems_i[e])

        @pl.loop(0, b_per_w - NBUF)
        def _(e):
            ecast = jnp.full((16,), e, jnp.int32)
            u16 = plsc.load_gather(uidx_v, [ecast])
            i16 = plsc.load_gather(iidx_v, [ecast])
            ncast = jnp.full((16,), e + NBUF, jnp.int32)
            un16 = plsc.load_gather(uidx_v, [ncast])
            in16 = plsc.load_gather(iidx_v, [ncast])
            for i in range(NBUF):
                @pl.when((e & (NBUF - 1)) == i)
                def _():
                    extract(e, u16 & (LANES - 1), stage_u[i], sems_u[i], 0)
                    fire(ut_hbm, un16[0], stage_u[i], sems_u[i])
                    extract(e, i16 & (LANES - 1), stage_i[i], sems_i[i],
                            xoff_i)
                    fire(it_hbm, in16[0], stage_i[i], sems_i[i])

        for e in range(b_per_w - NBUF, b_per_w):
            i = e % NBUF
            ecast = jnp.full((16,), e, jnp.int32)
            u16 = plsc.load_gather(uidx_v, [ecast])
            i16 = plsc.load_gather(iidx_v, [ecast])
            extract(e, u16 & (LANES - 1), stage_u[i], sems_u[i], 0)
            extract(e, i16 & (LANES - 1), stage_i[i], sems_i[i], xoff_i)

        for c in range(2 * EMBED):
            pltpu.sync_copy(
                xt_v.at[pl.ds(c * b_per_w, b_per_w)],
                xt_hbm.at[pl.ds(c * BATCH + base, b_per_w)])

    return gather_k(user_id, item_id, ut_t, it_t)


def _mlp_body(xt_ref, w1_ref, b1_ref, w2_ref, b2_ref, out_ref):
    h = jnp.dot(w1_ref[...], xt_ref[...], preferred_element_type=jnp.float32)
    h = jnp.maximum(h + b1_ref[...], 0.0)
    out_ref[...] = (
        jnp.dot(w2_ref[...], h, preferred_element_type=jnp.float32)
        + b2_ref[0, 0]
    )


def _tc_mlp(xt, W1, b1_col, W2, b2_2d):
    grid = (BATCH // BLK,)
    return pl.pallas_call(
        _mlp_body,
        grid=grid,
        in_specs=[
            pl.BlockSpec((2 * EMBED, BLK), lambda i: (0, i)),
            pl.BlockSpec((HIDDEN, 2 * EMBED), lambda i: (0, 0)),
            pl.BlockSpec((HIDDEN, 1), lambda i: (0, 0)),
            pl.BlockSpec((1, HIDDEN), lambda i: (0, 0)),
            pl.BlockSpec((1, 1), lambda i: (0, 0)),
        ],
        out_specs=pl.BlockSpec((1, BLK), lambda i: (0, i)),
        out_shape=jax.ShapeDtypeStruct((1, BATCH), jnp.float32),
    )(xt, W1, b1_col, W2, b2_2d)


def kernel(user_id, item_id, user_table, item_table, W1, b1, W2, b2):
    uid = user_id.astype(jnp.int32)
    iid = item_id.astype(jnp.int32)
    xt_flat = _sc_gather(uid, iid, user_table.T, item_table.T)
    xt = xt_flat.reshape(2 * EMBED, BATCH)
    out = _tc_mlp(xt, W1, b1.reshape(HIDDEN, 1), W2, b2.reshape(1, 1))
    return out.reshape(BATCH)
